# Initial kernel scaffold; baseline (speedup 1.0000x reference)
import jax
import jax.numpy as jnp
from jax.experimental import pallas as pl


def kernel(x, w_gate, w1, w2):
    raise NotImplementedError("write your pallas kernel here")


# speedup vs baseline: 1.5962x; 1.5962x over previous
import functools

import jax
import jax.numpy as jnp
from jax import lax
from jax.experimental import pallas as pl
from jax.experimental.pallas import tpu as pltpu
from jax.experimental.pallas import tpu_sc as plsc

E = 8
K = 2
D = 1024
F = 2048
T = 2048
C = 640
NROWS = (E + 1) * C
DUMP = E * C
LANES = 128
NW = 32
CHUNK = 64
DCH = 32


def _router_body(x_ref, wg_ref, d0_ref, d1_ref, wrow_ref,
                 counts_ref, aux_ref):
    x = x_ref[...]
    wg = wg_ref[...]
    logits = jnp.dot(x, wg, preferred_element_type=jnp.float32)
    lane = lax.broadcasted_iota(jnp.int32, (T, LANES), 1).astype(jnp.float32)
    valid = lane < float(E)
    m = jnp.max(jnp.where(valid, logits, -jnp.inf), axis=1, keepdims=True)
    ex = jnp.where(valid, jnp.exp(logits - m), 0.0)
    z = jnp.sum(ex, axis=1, keepdims=True)
    probs = ex / z

    m1 = jnp.max(probs, axis=1, keepdims=True)
    is1 = jnp.logical_and(probs == m1, valid)
    i1 = jnp.min(jnp.where(is1, lane, float(LANES)), axis=1, keepdims=True)
    mask0 = (lane == i1).astype(jnp.float32)
    p2 = jnp.where(mask0 > 0, -1.0, probs)
    m2 = jnp.max(p2, axis=1, keepdims=True)
    is2 = jnp.logical_and(p2 == m2, valid)
    i2 = jnp.min(jnp.where(is2, lane, float(LANES)), axis=1, keepdims=True)
    mask1 = (lane == i2).astype(jnp.float32)

    denom = m1 + m2 + 1e-9
    g0 = m1 / denom
    g1 = m2 / denom

    B = 256
    r = lax.broadcasted_iota(jnp.int32, (B, B), 0)
    c = lax.broadcasted_iota(jnp.int32, (B, B), 1)
    ltri = (r > c).astype(jnp.float32)
    carry = jnp.zeros((1, LANES), dtype=jnp.float32)
    pos_parts = []
    for mask in (mask0, mask1):
        parts = []
        for b in range(T // B):
            mb = mask[b * B:(b + 1) * B, :]
            parts.append(jnp.dot(ltri, mb, preferred_element_type=jnp.float32)
                         + carry)
            carry = carry + jnp.sum(mb, axis=0, keepdims=True)
        pos_parts.append(jnp.concatenate(parts, axis=0))
    pos0, pos1 = pos_parts
    total = carry

    p0 = jnp.sum(pos0 * mask0, axis=1, keepdims=True)
    p1 = jnp.sum(pos1 * mask1, axis=1, keepdims=True)
    keep0 = p0 < float(C)
    keep1 = p1 < float(C)
    d0 = jnp.where(keep0, i1 * float(C) + p0, float(DUMP))
    d1 = jnp.where(keep1, i2 * float(C) + p1, float(DUMP))
    d0_ref[...] = d0.astype(jnp.int32)
    d1_ref[...] = d1.astype(jnp.int32)
    w0e = jnp.where(keep0, g0, 0.0)
    w1e = jnp.where(keep1, g1, 0.0)
    w_all = jnp.concatenate([w0e, w1e], axis=0)
    wrow_ref[...] = jnp.broadcast_to(w_all, (K * T, 128))
    counts_ref[...] = jnp.minimum(total, float(C))

    em = jnp.maximum(mask0, mask1)
    tpe = jnp.sum(em, axis=0, keepdims=True) * (1.0 / T)
    ppe = jnp.sum(probs, axis=0, keepdims=True) * (1.0 / T)
    aux_ref[...] = jnp.sum(tpe * ppe, axis=1, keepdims=True) * float(E)


def _run_router(x, wg_pad, interpret=False):
    out_shapes = (
        jax.ShapeDtypeStruct((T, 1), jnp.int32),
        jax.ShapeDtypeStruct((T, 1), jnp.int32),
        jax.ShapeDtypeStruct((K * T, 128), jnp.float32),
        jax.ShapeDtypeStruct((1, LANES), jnp.float32),
        jax.ShapeDtypeStruct((1, 1), jnp.float32),
    )
    return pl.pallas_call(
        _router_body,
        out_shape=out_shapes,
        interpret=interpret,
    )(x, wg_pad)


def _dispatch_body(x_hbm, d_hbm, wbb_hbm, ein_hbm, wslot_hbm,
                   idx_v, rows_v, wrows_v, sem, sem2):
    cid = lax.axis_index("c")
    sid = lax.axis_index("s")
    w = sid * 2 + cid
    t0 = (w % 16) * (T // 16)
    pltpu.sync_copy(d_hbm.at[w], idx_v)
    pltpu.sync_copy(wbb_hbm.at[w], wrows_v)
    for j in range(2):
        pltpu.sync_copy(x_hbm.at[pl.ds(t0 + j * CHUNK, CHUNK)], rows_v)
        cp1 = pltpu.async_copy(rows_v, ein_hbm.at[idx_v.at[j]], sem)
        cp2 = pltpu.async_copy(wrows_v.at[j], wslot_hbm.at[idx_v.at[j]], sem2)
        cp1.wait()
        cp2.wait()


def _run_dispatch(x, d_b, w_bb):
    mesh = plsc.VectorSubcoreMesh(core_axis_name="c", subcore_axis_name="s")
    kern = functools.partial(
        pl.kernel,
        out_type=(
            jax.ShapeDtypeStruct((NROWS, D), jnp.float32),
            jax.ShapeDtypeStruct((NROWS, 128), jnp.float32),
        ),
        mesh=mesh,
        scratch_types=[
            pltpu.VMEM((2, CHUNK), jnp.int32),
            pltpu.VMEM((CHUNK, D), jnp.float32),
            pltpu.VMEM((2, CHUNK, 128), jnp.float32),
            pltpu.SemaphoreType.DMA,
            pltpu.SemaphoreType.DMA,
        ],
    )
    return kern(_dispatch_body)(x, d_b, w_bb)


def _ffn_body(counts_ref, xin_ref, w1_ref, w2_ref, ws_ref, out_ref):
    e = pl.program_id(0)
    cnt = counts_ref[0, e]
    row = lax.broadcasted_iota(jnp.int32, (C, 1), 0).astype(jnp.float32)
    x = jnp.where(row < cnt, xin_ref[...], 0.0)
    h = jnp.dot(x, w1_ref[0], preferred_element_type=jnp.float32)
    h = jax.nn.gelu(h, approximate=True)
    out = jnp.dot(h, w2_ref[0], preferred_element_type=jnp.float32)
    out_ref[...] = out * ws_ref[:, 0:1]


def _run_ffn(counts, ein, w1, w2, wslot, interpret=False):
    nblk = NROWS // C
    grid = (nblk,)
    return pl.pallas_call(
        _ffn_body,
        grid=grid,
        in_specs=[
            pl.BlockSpec(memory_space=pltpu.SMEM),
            pl.BlockSpec((C, D), lambda i: (i, 0)),
            pl.BlockSpec((1, D, F), lambda i: (jnp.minimum(i, E - 1), 0, 0)),
            pl.BlockSpec((1, F, D), lambda i: (jnp.minimum(i, E - 1), 0, 0)),
            pl.BlockSpec((C, 128), lambda i: (i, 0)),
        ],
        out_specs=pl.BlockSpec((C, D), lambda i: (i, 0)),
        out_shape=jax.ShapeDtypeStruct((NROWS, D), jnp.float32),
        interpret=interpret,
    )(counts, ein, w1, w2, wslot)


def _combine_body(outw_hbm, s_hbm, y_hbm, idx_v, bufa, bufb, sem):
    cid = lax.axis_index("c")
    sid = lax.axis_index("s")
    w = sid * 2 + cid
    t0 = w * (T // NW)
    pltpu.sync_copy(s_hbm.at[w], idx_v)
    for u in range(2):
        pltpu.async_copy(outw_hbm.at[idx_v.at[0, pl.ds(u * DCH, DCH)]],
                         bufa, sem).wait()
        pltpu.async_copy(outw_hbm.at[idx_v.at[1, pl.ds(u * DCH, DCH)]],
                         bufb, sem).wait()

        def rbody(r):
            for cc in range(D // 16):
                sl = pl.ds(cc * 16, 16)
                bufa[r, sl] = bufa[r, sl] + bufb[r, sl]
        pl.loop(0, DCH)(rbody)
        pltpu.sync_copy(bufa, y_hbm.at[pl.ds(t0 + u * DCH, DCH)])


def _run_combine(outw, s_d):
    mesh = plsc.VectorSubcoreMesh(core_axis_name="c", subcore_axis_name="s")
    kern = functools.partial(
        pl.kernel,
        out_type=jax.ShapeDtypeStruct((T, D), jnp.float32),
        mesh=mesh,
        scratch_types=[
            pltpu.VMEM((2, T // NW), jnp.int32),
            pltpu.VMEM((DCH, D), jnp.float32),
            pltpu.VMEM((DCH, D), jnp.float32),
            pltpu.SemaphoreType.DMA,
        ],
    )
    return kern(_combine_body)(outw, s_d)


def kernel(x, w_gate, w1, w2):
    wg_pad = jnp.pad(w_gate, ((0, 0), (0, LANES - E)))
    d0, d1, wrow, counts, aux = _run_router(x, wg_pad)

    dk = jnp.stack([d0[:, 0], d1[:, 0]])
    d_b = dk.reshape(2, 16, 2, CHUNK).reshape(NW, 2, CHUNK)
    w_bb = wrow.reshape(2, 16, 2, CHUNK, 128).reshape(NW, 2, CHUNK, 128)
    ein, wslot = _run_dispatch(x, d_b, w_bb)

    outw = _run_ffn(counts, ein, w1, w2, wslot)

    s_d = dk.reshape(2, NW, T // NW).transpose(1, 0, 2)
    y = _run_combine(outw, s_d)
    return y, aux[0, 0]
